# baseline (speedup 1.0000x reference)
import functools

import jax
import jax.numpy as jnp
from jax import lax
from jax.experimental import pallas as pl
from jax.experimental.pallas import tpu as pltpu
from jax.experimental.pallas import tpu_sc as plsc

N = 50000
E = 200000
D = 128
ODIM = 16

NC, NS, L = 2, 16, 16
NCHUNK = 8
CH = 6256
CH_LAST = N - 7 * CH
TRASH = CH
EPT = 12544
EHALF = EPT // 2
CAP_B = 100
ZR = 16
RPT = 400
CPT = 392
SROWS = NS * RPT

_mesh = plsc.VectorSubcoreMesh(core_axis_name="c", subcore_axis_name="s")


def _sc_agg_body(table, edges, agg, eseg, gsrc, gdst, rows3, rows4, zbuf,
                 acc, sem_ga):
    cid = lax.axis_index("c")
    sid = lax.axis_index("s")

    zv = jnp.zeros((L,), jnp.float32)

    def zrow(r, carry):
        for j in range(8):
            zbuf[r, pl.ds(j * L, L)] = zv
        return carry

    lax.fori_loop(0, ZR, zrow, 0)

    pltpu.sync_copy(edges.at[sid, 0], eseg)

    lane = lax.iota(jnp.int32, L)
    zero16 = jnp.zeros((L,), jnp.int32)
    pad_dst = jnp.full((L,), TRASH, jnp.int32)

    def chunk_body(ccc, carry0):
        chunk = cid * (NCHUNK // NC) + ccc
        base = pl.multiple_of(chunk * CH, 8)
        size = jnp.where(chunk == NCHUNK - 1, CH_LAST, CH)

        for z in range(RPT // ZR):
            off0 = pl.multiple_of(sid * RPT + z * ZR, 8)
            pltpu.sync_copy(zbuf, acc.at[pl.ds(off0, ZR)])
        plsc.subcore_barrier()

        def scan(i, off):
            v = eseg[pl.ds(pl.multiple_of(i * L, L), L)]
            local = (v & 0xFFFF) - base
            m = (local >= 0) & (local < size)
            mi = m.astype(jnp.int32)
            pos = off + plsc.cumsum(mi) - 1
            idx = [pos >> 7, zero16, pos & 127]
            plsc.store_scatter(gsrc, idx, lax.shift_right_logical(v, 16),
                               mask=m)
            plsc.store_scatter(gdst, idx, local, mask=m)
            return pos[L - 1] + 1

        off = lax.fori_loop(0, EPT // L, scan, jnp.int32(0))

        for j in range(16):
            p = off + j * L + lane
            idx = [p >> 7, zero16, p & 127]
            plsc.store_scatter(gsrc, idx, zero16)
            plsc.store_scatter(gdst, idx, pad_dst)
        nbp = (off + 255) >> 8

        def gbatch(i, carry):
            b0 = i * 2
            ga = pltpu.async_copy(table.at[gsrc.at[b0, 0]], rows3, sem_ga)
            gb = pltpu.async_copy(table.at[gsrc.at[b0 + 1, 0]], rows4, sem_ga)
            ga.wait()
            gb.wait()
            pltpu.sync_copy(rows3, acc.at[gdst.at[b0, 0]], add=True)
            pltpu.sync_copy(rows4, acc.at[gdst.at[b0 + 1, 0]], add=True)
            return carry

        lax.fori_loop(0, nbp, gbatch, 0)
        plsc.subcore_barrier()

        @pl.when(sid < NS - 1)
        def _():
            r0 = pl.multiple_of(sid * CPT, 8)
            pltpu.sync_copy(acc.at[pl.ds(r0, CPT)],
                            agg.at[pl.ds(base + r0, CPT)])

        @pl.when(jnp.logical_and(sid == NS - 1, chunk < NCHUNK - 1))
        def _():
            pltpu.sync_copy(acc.at[pl.ds(15 * CPT, CH - 15 * CPT)],
                            agg.at[pl.ds(base + 15 * CPT, CH - 15 * CPT)])

        @pl.when(jnp.logical_and(sid == NS - 1, chunk == NCHUNK - 1))
        def _():
            pltpu.sync_copy(acc.at[pl.ds(15 * CPT, CH_LAST - 15 * CPT)],
                            agg.at[pl.ds(base + 15 * CPT, CH_LAST - 15 * CPT)])

        plsc.subcore_barrier()
        return carry0

    lax.fori_loop(0, NCHUNK // NC, chunk_body, 0)


_sc_agg = pl.kernel(
    _sc_agg_body,
    out_type=jax.ShapeDtypeStruct((N, D), jnp.float32),
    mesh=_mesh,
    compiler_params=pltpu.CompilerParams(needs_layout_passes=False),
    scratch_types=[
        pltpu.VMEM((EPT,), jnp.int32),
        pltpu.VMEM((CAP_B, 1, 128), jnp.int32),
        pltpu.VMEM((CAP_B, 1, 128), jnp.int32),
        pltpu.VMEM((128, D), jnp.float32),
        pltpu.VMEM((128, D), jnp.float32),
        pltpu.VMEM((ZR, D), jnp.float32),
        pltpu.VMEM_SHARED((SROWS, D), jnp.float32),
        pltpu.SemaphoreType.DMA,
    ],
)


def _prep_edges(edge_index):
    packed = (edge_index[0] << 16) | edge_index[1]
    packed = jnp.concatenate(
        [packed, jnp.full((NS * EPT - E,), 0xFFFF, jnp.int32)])
    return packed.reshape(NS, 1, EPT)


BR = 2000


def _fc_body(x_ref, w_ref, b_ref, o_ref):
    o_ref[...] = jnp.maximum(
        jnp.dot(x_ref[...], w_ref[...], preferred_element_type=jnp.float32)
        + b_ref[...], 0.0)


def _gin2_body(x_ref, a1_ref, a2_ref, w1_ref, b1_ref, e1_ref, w2_ref, b2_ref,
               e2_ref, o_ref):
    x = x_ref[...]
    h1 = (1.0 + e1_ref[0, 0]) * x + a1_ref[...]
    h2 = (1.0 + e2_ref[0, 0]) * x + a2_ref[...]
    o1 = jnp.dot(h1, w1_ref[...], preferred_element_type=jnp.float32) + b1_ref[...]
    o2 = jnp.dot(h2, w2_ref[...], preferred_element_type=jnp.float32) + b2_ref[...]
    o_ref[...] = jnp.maximum(o1, 0.0) + jnp.maximum(o2, 0.0)


def _gin1_body(x_ref, a_ref, w_ref, b_ref, e_ref, o_ref):
    h = (1.0 + e_ref[0, 0]) * x_ref[...] + a_ref[...]
    o_ref[...] = jnp.maximum(
        jnp.dot(h, w_ref[...], preferred_element_type=jnp.float32)
        + b_ref[...], 0.0)


def _out_body(x_ref, w_ref, b_ref, o_ref):
    logits = (jnp.dot(x_ref[...], w_ref[...], preferred_element_type=jnp.float32)
              + b_ref[...])
    m = jnp.max(logits, axis=1, keepdims=True)
    e = jnp.exp(logits - m)
    o_ref[...] = e / jnp.sum(e, axis=1, keepdims=True)


_row_spec = pl.BlockSpec((BR, D), lambda i: (i, 0))
_w_spec = pl.BlockSpec((D, D), lambda i: (0, 0))
_b_spec = pl.BlockSpec((1, D), lambda i: (0, 0))
_s_spec = pl.BlockSpec((1, 1), lambda i: (0, 0))
_grid = (N // BR,)


def _fc(x, w, b):
    return pl.pallas_call(
        _fc_body,
        grid=_grid,
        in_specs=[_row_spec, _w_spec, _b_spec],
        out_specs=_row_spec,
        out_shape=jax.ShapeDtypeStruct((N, D), jnp.float32),
    )(x, w, b.reshape(1, D))


def _gin2(x, a1, a2, w1, b1, e1, w2, b2, e2):
    return pl.pallas_call(
        _gin2_body,
        grid=_grid,
        in_specs=[_row_spec, _row_spec, _row_spec, _w_spec, _b_spec, _s_spec,
                  _w_spec, _b_spec, _s_spec],
        out_specs=_row_spec,
        out_shape=jax.ShapeDtypeStruct((N, D), jnp.float32),
    )(x, a1, a2, w1, b1.reshape(1, D), e1.reshape(1, 1), w2, b2.reshape(1, D),
      e2.reshape(1, 1))


def _gin1(x, a, w, b, e):
    return pl.pallas_call(
        _gin1_body,
        grid=_grid,
        in_specs=[_row_spec, _row_spec, _w_spec, _b_spec, _s_spec],
        out_specs=_row_spec,
        out_shape=jax.ShapeDtypeStruct((N, D), jnp.float32),
    )(x, a, w, b.reshape(1, D), e.reshape(1, 1))


def _head(x, w, b):
    return pl.pallas_call(
        _out_body,
        grid=_grid,
        in_specs=[_row_spec, pl.BlockSpec((D, ODIM), lambda i: (0, 0)),
                  pl.BlockSpec((1, ODIM), lambda i: (0, 0))],
        out_specs=pl.BlockSpec((BR, ODIM), lambda i: (i, 0)),
        out_shape=jax.ShapeDtypeStruct((N, ODIM), jnp.float32),
    )(x, w, b.reshape(1, ODIM))


def kernel(x_glomeruli, x_cell, edge_index_gg, edge_index_cg, edge_index_cc,
           W_fc_g, b_fc_g, W_fc_c, b_fc_c,
           W0_gg, b0_gg, eps0_gg, W0_cg, b0_cg, eps0_cg, W0_cc, b0_cc, eps0_cc,
           W1_gg, b1_gg, eps1_gg, W1_cg, b1_cg, eps1_cg, W1_cc, b1_cc, eps1_cc,
           W_out, b_out):
    e_gg = _prep_edges(edge_index_gg)
    e_cg = _prep_edges(edge_index_cg)
    e_cc = _prep_edges(edge_index_cc)

    xg = _fc(x_glomeruli, W_fc_g, b_fc_g)
    xc = _fc(x_cell, W_fc_c, b_fc_c)

    agg_gg = _sc_agg(xg, e_gg)
    agg_cg = _sc_agg(xc, e_cg)
    agg_cc = _sc_agg(xc, e_cc)
    xg1 = _gin2(xg, agg_gg, agg_cg, W0_gg, b0_gg, eps0_gg, W0_cg, b0_cg, eps0_cg)
    xc1 = _gin1(xc, agg_cc, W0_cc, b0_cc, eps0_cc)

    agg1_gg = _sc_agg(xg1, e_gg)
    agg1_cg = _sc_agg(xc1, e_cg)
    xg2 = _gin2(xg1, agg1_gg, agg1_cg, W1_gg, b1_gg, eps1_gg, W1_cg, b1_cg,
                eps1_cg)

    return _head(xg2, W_out, b_out)


# speedup vs baseline: 2.2645x; 2.2645x over previous
import functools

import jax
import jax.numpy as jnp
from jax import lax
from jax.experimental import pallas as pl
from jax.experimental.pallas import tpu as pltpu
from jax.experimental.pallas import tpu_sc as plsc

N = 50000
E = 200000
D = 128
ODIM = 16

NC, NS, L = 2, 16, 16
NCHUNK = 6
CH = 8336
CH_LAST = N - 5 * CH
TRASH = CH
EPT = 12544
EHALF = EPT // 2
CAP_B = 100
ZR = 16
RPT = 528
CPT = 520
SROWS = NS * RPT

_mesh = plsc.VectorSubcoreMesh(core_axis_name="c", subcore_axis_name="s")


def _sc_agg_body(table, edges, agg, eseg, gsrc, gdst, rows3, zbuf, acc,
                 sem_ga):
    cid = lax.axis_index("c")
    sid = lax.axis_index("s")

    zv = jnp.zeros((L,), jnp.float32)

    def zrow(r, carry):
        for j in range(8):
            zbuf[r, pl.ds(j * L, L)] = zv
        return carry

    lax.fori_loop(0, ZR, zrow, 0)

    pltpu.sync_copy(edges.at[sid, 0], eseg)

    lane = lax.iota(jnp.int32, L)
    zero16 = jnp.zeros((L,), jnp.int32)
    pad_dst = jnp.full((L,), TRASH, jnp.int32)

    def chunk_body(ccc, carry0):
        chunk = cid * (NCHUNK // NC) + ccc
        base = pl.multiple_of(chunk * CH, 8)
        size = jnp.where(chunk == NCHUNK - 1, CH_LAST, CH)

        for z in range(RPT // ZR):
            off0 = pl.multiple_of(sid * RPT + z * ZR, 8)
            pltpu.sync_copy(zbuf, acc.at[pl.ds(off0, ZR)])
        plsc.subcore_barrier()

        def scan(i, off):
            v = eseg[pl.ds(pl.multiple_of(i * L, L), L)]
            local = (v & 0xFFFF) - base
            m = (local >= 0) & (local < size)
            mi = m.astype(jnp.int32)
            pos = off + plsc.cumsum(mi) - 1
            idx = [pos >> 7, zero16, pos & 127]
            plsc.store_scatter(gsrc, idx, lax.shift_right_logical(v, 16),
                               mask=m)
            plsc.store_scatter(gdst, idx, local, mask=m)
            return pos[L - 1] + 1

        off = lax.fori_loop(0, EPT // L, scan, jnp.int32(0))

        for j in range(8):
            p = off + j * L + lane
            idx = [p >> 7, zero16, p & 127]
            plsc.store_scatter(gsrc, idx, zero16)
            plsc.store_scatter(gdst, idx, pad_dst)
        nb = (off + 127) >> 7

        def gbatch(b, carry):
            pltpu.async_copy(table.at[gsrc.at[b, 0]], rows3, sem_ga).wait()
            pltpu.sync_copy(rows3, acc.at[gdst.at[b, 0]], add=True)
            return carry

        lax.fori_loop(0, nb, gbatch, 0)
        plsc.subcore_barrier()

        @pl.when(sid < NS - 1)
        def _():
            r0 = pl.multiple_of(sid * CPT, 8)
            pltpu.sync_copy(acc.at[pl.ds(r0, CPT)],
                            agg.at[pl.ds(base + r0, CPT)])

        @pl.when(jnp.logical_and(sid == NS - 1, chunk < NCHUNK - 1))
        def _():
            pltpu.sync_copy(acc.at[pl.ds(15 * CPT, CH - 15 * CPT)],
                            agg.at[pl.ds(base + 15 * CPT, CH - 15 * CPT)])

        @pl.when(jnp.logical_and(sid == NS - 1, chunk == NCHUNK - 1))
        def _():
            pltpu.sync_copy(acc.at[pl.ds(15 * CPT, CH_LAST - 15 * CPT)],
                            agg.at[pl.ds(base + 15 * CPT, CH_LAST - 15 * CPT)])

        plsc.subcore_barrier()
        return carry0

    lax.fori_loop(0, NCHUNK // NC, chunk_body, 0)


_sc_agg = pl.kernel(
    _sc_agg_body,
    out_type=jax.ShapeDtypeStruct((N, D), jnp.float32),
    mesh=_mesh,
    compiler_params=pltpu.CompilerParams(needs_layout_passes=False),
    scratch_types=[
        pltpu.VMEM((EPT,), jnp.int32),
        pltpu.VMEM((CAP_B, 1, 128), jnp.int32),
        pltpu.VMEM((CAP_B, 1, 128), jnp.int32),
        pltpu.VMEM((128, D), jnp.float32),
        pltpu.VMEM((ZR, D), jnp.float32),
        pltpu.VMEM_SHARED((SROWS, D), jnp.float32),
        pltpu.SemaphoreType.DMA,
    ],
)


def _prep_edges(edge_index):
    packed = (edge_index[0] << 16) | edge_index[1]
    packed = jnp.concatenate(
        [packed, jnp.full((NS * EPT - E,), 0xFFFF, jnp.int32)])
    return packed.reshape(NS, 1, EPT)


BR = 2000


def _fc_body(x_ref, w_ref, b_ref, o_ref):
    o_ref[...] = jnp.maximum(
        jnp.dot(x_ref[...], w_ref[...], preferred_element_type=jnp.float32)
        + b_ref[...], 0.0)


def _gin2_body(x_ref, a1_ref, a2_ref, w1_ref, b1_ref, e1_ref, w2_ref, b2_ref,
               e2_ref, o_ref):
    x = x_ref[...]
    h1 = (1.0 + e1_ref[0, 0]) * x + a1_ref[...]
    h2 = (1.0 + e2_ref[0, 0]) * x + a2_ref[...]
    o1 = jnp.dot(h1, w1_ref[...], preferred_element_type=jnp.float32) + b1_ref[...]
    o2 = jnp.dot(h2, w2_ref[...], preferred_element_type=jnp.float32) + b2_ref[...]
    o_ref[...] = jnp.maximum(o1, 0.0) + jnp.maximum(o2, 0.0)


def _gin1_body(x_ref, a_ref, w_ref, b_ref, e_ref, o_ref):
    h = (1.0 + e_ref[0, 0]) * x_ref[...] + a_ref[...]
    o_ref[...] = jnp.maximum(
        jnp.dot(h, w_ref[...], preferred_element_type=jnp.float32)
        + b_ref[...], 0.0)


def _out_body(x_ref, w_ref, b_ref, o_ref):
    logits = (jnp.dot(x_ref[...], w_ref[...], preferred_element_type=jnp.float32)
              + b_ref[...])
    m = jnp.max(logits, axis=1, keepdims=True)
    e = jnp.exp(logits - m)
    o_ref[...] = e / jnp.sum(e, axis=1, keepdims=True)


_row_spec = pl.BlockSpec((BR, D), lambda i: (i, 0))
_w_spec = pl.BlockSpec((D, D), lambda i: (0, 0))
_b_spec = pl.BlockSpec((1, D), lambda i: (0, 0))
_s_spec = pl.BlockSpec((1, 1), lambda i: (0, 0))
_grid = (N // BR,)


def _fc(x, w, b):
    return pl.pallas_call(
        _fc_body,
        grid=_grid,
        in_specs=[_row_spec, _w_spec, _b_spec],
        out_specs=_row_spec,
        out_shape=jax.ShapeDtypeStruct((N, D), jnp.float32),
    )(x, w, b.reshape(1, D))


def _gin2(x, a1, a2, w1, b1, e1, w2, b2, e2):
    return pl.pallas_call(
        _gin2_body,
        grid=_grid,
        in_specs=[_row_spec, _row_spec, _row_spec, _w_spec, _b_spec, _s_spec,
                  _w_spec, _b_spec, _s_spec],
        out_specs=_row_spec,
        out_shape=jax.ShapeDtypeStruct((N, D), jnp.float32),
    )(x, a1, a2, w1, b1.reshape(1, D), e1.reshape(1, 1), w2, b2.reshape(1, D),
      e2.reshape(1, 1))


def _gin1(x, a, w, b, e):
    return pl.pallas_call(
        _gin1_body,
        grid=_grid,
        in_specs=[_row_spec, _row_spec, _w_spec, _b_spec, _s_spec],
        out_specs=_row_spec,
        out_shape=jax.ShapeDtypeStruct((N, D), jnp.float32),
    )(x, a, w, b.reshape(1, D), e.reshape(1, 1))


def _head(x, w, b):
    return pl.pallas_call(
        _out_body,
        grid=_grid,
        in_specs=[_row_spec, pl.BlockSpec((D, ODIM), lambda i: (0, 0)),
                  pl.BlockSpec((1, ODIM), lambda i: (0, 0))],
        out_specs=pl.BlockSpec((BR, ODIM), lambda i: (i, 0)),
        out_shape=jax.ShapeDtypeStruct((N, ODIM), jnp.float32),
    )(x, w, b.reshape(1, ODIM))


def kernel(x_glomeruli, x_cell, edge_index_gg, edge_index_cg, edge_index_cc,
           W_fc_g, b_fc_g, W_fc_c, b_fc_c,
           W0_gg, b0_gg, eps0_gg, W0_cg, b0_cg, eps0_cg, W0_cc, b0_cc, eps0_cc,
           W1_gg, b1_gg, eps1_gg, W1_cg, b1_cg, eps1_cg, W1_cc, b1_cc, eps1_cc,
           W_out, b_out):
    e_gg = _prep_edges(edge_index_gg)
    e_cg = _prep_edges(edge_index_cg)
    e_cc = _prep_edges(edge_index_cc)

    xg = _fc(x_glomeruli, W_fc_g, b_fc_g)
    xc = _fc(x_cell, W_fc_c, b_fc_c)

    agg_gg = _sc_agg(xg, e_gg)
    agg_cg = _sc_agg(xc, e_cg)
    agg_cc = _sc_agg(xc, e_cc)
    xg1 = _gin2(xg, agg_gg, agg_cg, W0_gg, b0_gg, eps0_gg, W0_cg, b0_cg, eps0_cg)
    xc1 = _gin1(xc, agg_cc, W0_cc, b0_cc, eps0_cc)

    agg1_gg = _sc_agg(xg1, e_gg)
    agg1_cg = _sc_agg(xc1, e_cg)
    xg2 = _gin2(xg1, agg1_gg, agg1_cg, W1_gg, b1_gg, eps1_gg, W1_cg, b1_cg,
                eps1_cg)

    return _head(xg2, W_out, b_out)


# speedup vs baseline: 2.2820x; 1.0077x over previous
import functools

import jax
import jax.numpy as jnp
from jax import lax
from jax.experimental import pallas as pl
from jax.experimental.pallas import tpu as pltpu
from jax.experimental.pallas import tpu_sc as plsc

N = 50000
E = 200000
D = 128
ODIM = 16

NC, NS, L = 2, 16, 16
NCHUNK = 6
CH = 8336
CH_LAST = N - 5 * CH
TRASH = CH
EPT = 12544
EHALF = EPT // 2
CAP_B = 100
ZR = 48
RPT = 528
CPT = 520
SROWS = NS * RPT

_mesh = plsc.VectorSubcoreMesh(core_axis_name="c", subcore_axis_name="s")


def _sc_agg_body(table, edges, agg, eseg, gsrc, gdst, rows3, zbuf, acc,
                 sem_ga):
    cid = lax.axis_index("c")
    sid = lax.axis_index("s")

    zv = jnp.zeros((L,), jnp.float32)

    def zrow(r, carry):
        for j in range(8):
            zbuf[r, pl.ds(j * L, L)] = zv
        return carry

    lax.fori_loop(0, ZR, zrow, 0)

    pltpu.sync_copy(edges.at[sid, 0], eseg)

    lane = lax.iota(jnp.int32, L)
    zero16 = jnp.zeros((L,), jnp.int32)
    pad_dst = jnp.full((L,), TRASH, jnp.int32)

    def chunk_body(ccc, carry0):
        chunk = cid * (NCHUNK // NC) + ccc
        base = pl.multiple_of(chunk * CH, 8)
        size = jnp.where(chunk == NCHUNK - 1, CH_LAST, CH)

        for z in range(RPT // ZR):
            off0 = pl.multiple_of(sid * RPT + z * ZR, 8)
            pltpu.sync_copy(zbuf, acc.at[pl.ds(off0, ZR)])
        plsc.subcore_barrier()

        def scan(i, off):
            v = eseg[pl.ds(pl.multiple_of(i * L, L), L)]
            local = (v & 0xFFFF) - base
            m = (local >= 0) & (local < size)
            mi = m.astype(jnp.int32)
            pos = off + plsc.cumsum(mi) - 1
            idx = [pos >> 7, zero16, pos & 127]
            plsc.store_scatter(gsrc, idx, lax.shift_right_logical(v, 16),
                               mask=m)
            plsc.store_scatter(gdst, idx, local, mask=m)
            return pos[L - 1] + 1

        off = lax.fori_loop(0, EPT // L, scan, jnp.int32(0))

        for j in range(8):
            p = off + j * L + lane
            idx = [p >> 7, zero16, p & 127]
            plsc.store_scatter(gsrc, idx, zero16)
            plsc.store_scatter(gdst, idx, pad_dst)
        nb = (off + 127) >> 7

        def gbatch(b, carry):
            pltpu.async_copy(table.at[gsrc.at[b, 0]], rows3, sem_ga).wait()
            pltpu.sync_copy(rows3, acc.at[gdst.at[b, 0]], add=True)
            return carry

        lax.fori_loop(0, nb, gbatch, 0)
        plsc.subcore_barrier()

        @pl.when(sid < NS - 1)
        def _():
            r0 = pl.multiple_of(sid * CPT, 8)
            pltpu.sync_copy(acc.at[pl.ds(r0, CPT)],
                            agg.at[pl.ds(base + r0, CPT)])

        @pl.when(jnp.logical_and(sid == NS - 1, chunk < NCHUNK - 1))
        def _():
            pltpu.sync_copy(acc.at[pl.ds(15 * CPT, CH - 15 * CPT)],
                            agg.at[pl.ds(base + 15 * CPT, CH - 15 * CPT)])

        @pl.when(jnp.logical_and(sid == NS - 1, chunk == NCHUNK - 1))
        def _():
            pltpu.sync_copy(acc.at[pl.ds(15 * CPT, CH_LAST - 15 * CPT)],
                            agg.at[pl.ds(base + 15 * CPT, CH_LAST - 15 * CPT)])

        plsc.subcore_barrier()
        return carry0

    lax.fori_loop(0, NCHUNK // NC, chunk_body, 0)


_sc_agg = pl.kernel(
    _sc_agg_body,
    out_type=jax.ShapeDtypeStruct((N, D), jnp.float32),
    mesh=_mesh,
    compiler_params=pltpu.CompilerParams(needs_layout_passes=False),
    scratch_types=[
        pltpu.VMEM((EPT,), jnp.int32),
        pltpu.VMEM((CAP_B, 1, 128), jnp.int32),
        pltpu.VMEM((CAP_B, 1, 128), jnp.int32),
        pltpu.VMEM((128, D), jnp.float32),
        pltpu.VMEM((ZR, D), jnp.float32),
        pltpu.VMEM_SHARED((SROWS, D), jnp.float32),
        pltpu.SemaphoreType.DMA,
    ],
)


def _prep_edges(edge_index):
    packed = (edge_index[0] << 16) | edge_index[1]
    packed = jnp.concatenate(
        [packed, jnp.full((NS * EPT - E,), 0xFFFF, jnp.int32)])
    return packed.reshape(NS, 1, EPT)


BR = 2000


def _fc_body(x_ref, w_ref, b_ref, o_ref):
    o_ref[...] = jnp.maximum(
        jnp.dot(x_ref[...], w_ref[...], preferred_element_type=jnp.float32)
        + b_ref[...], 0.0)


def _gin2_body(x_ref, a1_ref, a2_ref, w1_ref, b1_ref, e1_ref, w2_ref, b2_ref,
               e2_ref, o_ref):
    x = x_ref[...]
    h1 = (1.0 + e1_ref[0, 0]) * x + a1_ref[...]
    h2 = (1.0 + e2_ref[0, 0]) * x + a2_ref[...]
    o1 = jnp.dot(h1, w1_ref[...], preferred_element_type=jnp.float32) + b1_ref[...]
    o2 = jnp.dot(h2, w2_ref[...], preferred_element_type=jnp.float32) + b2_ref[...]
    o_ref[...] = jnp.maximum(o1, 0.0) + jnp.maximum(o2, 0.0)


def _gin1_body(x_ref, a_ref, w_ref, b_ref, e_ref, o_ref):
    h = (1.0 + e_ref[0, 0]) * x_ref[...] + a_ref[...]
    o_ref[...] = jnp.maximum(
        jnp.dot(h, w_ref[...], preferred_element_type=jnp.float32)
        + b_ref[...], 0.0)


def _out_body(x_ref, w_ref, b_ref, o_ref):
    logits = (jnp.dot(x_ref[...], w_ref[...], preferred_element_type=jnp.float32)
              + b_ref[...])
    m = jnp.max(logits, axis=1, keepdims=True)
    e = jnp.exp(logits - m)
    o_ref[...] = e / jnp.sum(e, axis=1, keepdims=True)


_row_spec = pl.BlockSpec((BR, D), lambda i: (i, 0))
_w_spec = pl.BlockSpec((D, D), lambda i: (0, 0))
_b_spec = pl.BlockSpec((1, D), lambda i: (0, 0))
_s_spec = pl.BlockSpec((1, 1), lambda i: (0, 0))
_grid = (N // BR,)


def _fc(x, w, b):
    return pl.pallas_call(
        _fc_body,
        grid=_grid,
        in_specs=[_row_spec, _w_spec, _b_spec],
        out_specs=_row_spec,
        out_shape=jax.ShapeDtypeStruct((N, D), jnp.float32),
    )(x, w, b.reshape(1, D))


def _gin2(x, a1, a2, w1, b1, e1, w2, b2, e2):
    return pl.pallas_call(
        _gin2_body,
        grid=_grid,
        in_specs=[_row_spec, _row_spec, _row_spec, _w_spec, _b_spec, _s_spec,
                  _w_spec, _b_spec, _s_spec],
        out_specs=_row_spec,
        out_shape=jax.ShapeDtypeStruct((N, D), jnp.float32),
    )(x, a1, a2, w1, b1.reshape(1, D), e1.reshape(1, 1), w2, b2.reshape(1, D),
      e2.reshape(1, 1))


def _gin1(x, a, w, b, e):
    return pl.pallas_call(
        _gin1_body,
        grid=_grid,
        in_specs=[_row_spec, _row_spec, _w_spec, _b_spec, _s_spec],
        out_specs=_row_spec,
        out_shape=jax.ShapeDtypeStruct((N, D), jnp.float32),
    )(x, a, w, b.reshape(1, D), e.reshape(1, 1))


def _head(x, w, b):
    return pl.pallas_call(
        _out_body,
        grid=_grid,
        in_specs=[_row_spec, pl.BlockSpec((D, ODIM), lambda i: (0, 0)),
                  pl.BlockSpec((1, ODIM), lambda i: (0, 0))],
        out_specs=pl.BlockSpec((BR, ODIM), lambda i: (i, 0)),
        out_shape=jax.ShapeDtypeStruct((N, ODIM), jnp.float32),
    )(x, w, b.reshape(1, ODIM))


def kernel(x_glomeruli, x_cell, edge_index_gg, edge_index_cg, edge_index_cc,
           W_fc_g, b_fc_g, W_fc_c, b_fc_c,
           W0_gg, b0_gg, eps0_gg, W0_cg, b0_cg, eps0_cg, W0_cc, b0_cc, eps0_cc,
           W1_gg, b1_gg, eps1_gg, W1_cg, b1_cg, eps1_cg, W1_cc, b1_cc, eps1_cc,
           W_out, b_out):
    e_gg = _prep_edges(edge_index_gg)
    e_cg = _prep_edges(edge_index_cg)
    e_cc = _prep_edges(edge_index_cc)

    xg = _fc(x_glomeruli, W_fc_g, b_fc_g)
    xc = _fc(x_cell, W_fc_c, b_fc_c)

    agg_gg = _sc_agg(xg, e_gg)
    agg_cg = _sc_agg(xc, e_cg)
    agg_cc = _sc_agg(xc, e_cc)
    xg1 = _gin2(xg, agg_gg, agg_cg, W0_gg, b0_gg, eps0_gg, W0_cg, b0_cg, eps0_cg)
    xc1 = _gin1(xc, agg_cc, W0_cc, b0_cc, eps0_cc)

    agg1_gg = _sc_agg(xg1, e_gg)
    agg1_cg = _sc_agg(xc1, e_cg)
    xg2 = _gin2(xg1, agg1_gg, agg1_cg, W1_gg, b1_gg, eps1_gg, W1_cg, b1_cg,
                eps1_cg)

    return _head(xg2, W_out, b_out)


# speedup vs baseline: 2.2829x; 1.0004x over previous
import jax
import jax.numpy as jnp
from jax import lax
from jax.experimental import pallas as pl
from jax.experimental.pallas import tpu as pltpu
from jax.experimental.pallas import tpu_sc as plsc

N = 50000
E = 200000
D = 128
ODIM = 16

NC, NS, L = 2, 16, 16
NCHUNK = 6
CH = 8336
CH_LAST = N - 5 * CH
TRASH = CH
EPT = 12544
CAP_B = 100
ZR = 48
RPT = 528
CPT = 520
SROWS = NS * RPT

_mesh = plsc.VectorSubcoreMesh(core_axis_name="c", subcore_axis_name="s")


def _sc_agg_body(table, edges, agg, eseg, gsrc, gdst, rows3, zbuf, acc,
                 sem_ga):
    cid = lax.axis_index("c")
    sid = lax.axis_index("s")

    zv = jnp.zeros((L,), jnp.float32)

    def zrow(r, carry):
        for j in range(8):
            zbuf[r, pl.ds(j * L, L)] = zv
        return carry

    lax.fori_loop(0, ZR, zrow, 0)

    pltpu.sync_copy(edges.at[sid, 0], eseg)

    lane = lax.iota(jnp.int32, L)
    zero16 = jnp.zeros((L,), jnp.int32)
    pad_dst = jnp.full((L,), TRASH, jnp.int32)

    def chunk_body(ccc, carry0):
        chunk = cid * (NCHUNK // NC) + ccc
        base = pl.multiple_of(chunk * CH, 8)
        size = jnp.where(chunk == NCHUNK - 1, CH_LAST, CH)

        for z in range(RPT // ZR):
            off0 = pl.multiple_of(sid * RPT + z * ZR, 8)
            pltpu.sync_copy(zbuf, acc.at[pl.ds(off0, ZR)])
        plsc.subcore_barrier()

        def scan(i, off):
            v = eseg[pl.ds(pl.multiple_of(i * L, L), L)]
            local = (v & 0xFFFF) - base
            m = (local >= 0) & (local < size)
            mi = m.astype(jnp.int32)
            pos = off + plsc.cumsum(mi) - 1
            idx = [pos >> 7, zero16, pos & 127]
            plsc.store_scatter(gsrc, idx, lax.shift_right_logical(v, 16),
                               mask=m)
            plsc.store_scatter(gdst, idx, local, mask=m)
            return pos[L - 1] + 1

        off = lax.fori_loop(0, EPT // L, scan, jnp.int32(0))

        for j in range(8):
            p = off + j * L + lane
            idx = [p >> 7, zero16, p & 127]
            plsc.store_scatter(gsrc, idx, zero16)
            plsc.store_scatter(gdst, idx, pad_dst)
        nb = (off + 127) >> 7

        def gbatch(b, carry):
            pltpu.async_copy(table.at[gsrc.at[b, 0]], rows3, sem_ga).wait()
            pltpu.sync_copy(rows3, acc.at[gdst.at[b, 0]], add=True)
            return carry

        lax.fori_loop(0, nb, gbatch, 0)
        plsc.subcore_barrier()

        @pl.when(sid < NS - 1)
        def _():
            r0 = pl.multiple_of(sid * CPT, 8)
            pltpu.sync_copy(acc.at[pl.ds(r0, CPT)],
                            agg.at[pl.ds(base + r0, CPT)])

        @pl.when(jnp.logical_and(sid == NS - 1, chunk < NCHUNK - 1))
        def _():
            pltpu.sync_copy(acc.at[pl.ds(15 * CPT, CH - 15 * CPT)],
                            agg.at[pl.ds(base + 15 * CPT, CH - 15 * CPT)])

        @pl.when(jnp.logical_and(sid == NS - 1, chunk == NCHUNK - 1))
        def _():
            pltpu.sync_copy(acc.at[pl.ds(15 * CPT, CH_LAST - 15 * CPT)],
                            agg.at[pl.ds(base + 15 * CPT, CH_LAST - 15 * CPT)])

        plsc.subcore_barrier()
        return carry0

    lax.fori_loop(0, NCHUNK // NC, chunk_body, 0)


_sc_agg = pl.kernel(
    _sc_agg_body,
    out_type=jax.ShapeDtypeStruct((N, D), jnp.float32),
    mesh=_mesh,
    compiler_params=pltpu.CompilerParams(needs_layout_passes=False),
    scratch_types=[
        pltpu.VMEM((EPT,), jnp.int32),
        pltpu.VMEM((CAP_B, 1, 128), jnp.int32),
        pltpu.VMEM((CAP_B, 1, 128), jnp.int32),
        pltpu.VMEM((128, D), jnp.float32),
        pltpu.VMEM((ZR, D), jnp.float32),
        pltpu.VMEM_SHARED((SROWS, D), jnp.float32),
        pltpu.SemaphoreType.DMA,
    ],
)


def _prep_edges(edge_index):
    packed = (edge_index[0] << 16) | edge_index[1]
    packed = jnp.concatenate(
        [packed, jnp.full((NS * EPT - E,), 0xFFFF, jnp.int32)])
    return packed.reshape(NS, 1, EPT)


BR = 2000


def _fc_body(x_ref, w_ref, b_ref, o_ref):
    o_ref[...] = jnp.maximum(
        jnp.dot(x_ref[...], w_ref[...], preferred_element_type=jnp.float32)
        + b_ref[...], 0.0)


def _gin2_body(x_ref, a1_ref, a2_ref, w1_ref, b1_ref, e1_ref, w2_ref, b2_ref,
               e2_ref, o_ref):
    x = x_ref[...]
    h1 = (1.0 + e1_ref[0, 0]) * x + a1_ref[...]
    h2 = (1.0 + e2_ref[0, 0]) * x + a2_ref[...]
    o1 = jnp.dot(h1, w1_ref[...], preferred_element_type=jnp.float32) + b1_ref[...]
    o2 = jnp.dot(h2, w2_ref[...], preferred_element_type=jnp.float32) + b2_ref[...]
    o_ref[...] = jnp.maximum(o1, 0.0) + jnp.maximum(o2, 0.0)


def _gin1_body(x_ref, a_ref, w_ref, b_ref, e_ref, o_ref):
    h = (1.0 + e_ref[0, 0]) * x_ref[...] + a_ref[...]
    o_ref[...] = jnp.maximum(
        jnp.dot(h, w_ref[...], preferred_element_type=jnp.float32)
        + b_ref[...], 0.0)


def _out_body(x_ref, w_ref, b_ref, o_ref):
    logits = (jnp.dot(x_ref[...], w_ref[...], preferred_element_type=jnp.float32)
              + b_ref[...])
    m = jnp.max(logits, axis=1, keepdims=True)
    e = jnp.exp(logits - m)
    o_ref[...] = e / jnp.sum(e, axis=1, keepdims=True)


_row_spec = pl.BlockSpec((BR, D), lambda i: (i, 0))
_w_spec = pl.BlockSpec((D, D), lambda i: (0, 0))
_b_spec = pl.BlockSpec((1, D), lambda i: (0, 0))
_s_spec = pl.BlockSpec((1, 1), lambda i: (0, 0))
_grid = (N // BR,)


def _fc(x, w, b):
    return pl.pallas_call(
        _fc_body,
        grid=_grid,
        in_specs=[_row_spec, _w_spec, _b_spec],
        out_specs=_row_spec,
        out_shape=jax.ShapeDtypeStruct((N, D), jnp.float32),
    )(x, w, b.reshape(1, D))


def _gin2(x, a1, a2, w1, b1, e1, w2, b2, e2):
    return pl.pallas_call(
        _gin2_body,
        grid=_grid,
        in_specs=[_row_spec, _row_spec, _row_spec, _w_spec, _b_spec, _s_spec,
                  _w_spec, _b_spec, _s_spec],
        out_specs=_row_spec,
        out_shape=jax.ShapeDtypeStruct((N, D), jnp.float32),
    )(x, a1, a2, w1, b1.reshape(1, D), e1.reshape(1, 1), w2, b2.reshape(1, D),
      e2.reshape(1, 1))


def _gin1(x, a, w, b, e):
    return pl.pallas_call(
        _gin1_body,
        grid=_grid,
        in_specs=[_row_spec, _row_spec, _w_spec, _b_spec, _s_spec],
        out_specs=_row_spec,
        out_shape=jax.ShapeDtypeStruct((N, D), jnp.float32),
    )(x, a, w, b.reshape(1, D), e.reshape(1, 1))


def _head(x, w, b):
    return pl.pallas_call(
        _out_body,
        grid=_grid,
        in_specs=[_row_spec, pl.BlockSpec((D, ODIM), lambda i: (0, 0)),
                  pl.BlockSpec((1, ODIM), lambda i: (0, 0))],
        out_specs=pl.BlockSpec((BR, ODIM), lambda i: (i, 0)),
        out_shape=jax.ShapeDtypeStruct((N, ODIM), jnp.float32),
    )(x, w, b.reshape(1, ODIM))


def kernel(x_glomeruli, x_cell, edge_index_gg, edge_index_cg, edge_index_cc,
           W_fc_g, b_fc_g, W_fc_c, b_fc_c,
           W0_gg, b0_gg, eps0_gg, W0_cg, b0_cg, eps0_cg, W0_cc, b0_cc, eps0_cc,
           W1_gg, b1_gg, eps1_gg, W1_cg, b1_cg, eps1_cg, W1_cc, b1_cc, eps1_cc,
           W_out, b_out):
    e_gg = _prep_edges(edge_index_gg)
    e_cg = _prep_edges(edge_index_cg)
    e_cc = _prep_edges(edge_index_cc)

    xg = _fc(x_glomeruli, W_fc_g, b_fc_g)
    xc = _fc(x_cell, W_fc_c, b_fc_c)

    agg_gg = _sc_agg(xg, e_gg)
    agg_cg = _sc_agg(xc, e_cg)
    agg_cc = _sc_agg(xc, e_cc)
    xg1 = _gin2(xg, agg_gg, agg_cg, W0_gg, b0_gg, eps0_gg, W0_cg, b0_cg, eps0_cg)
    xc1 = _gin1(xc, agg_cc, W0_cc, b0_cc, eps0_cc)

    agg1_gg = _sc_agg(xg1, e_gg)
    agg1_cg = _sc_agg(xc1, e_cg)
    xg2 = _gin2(xg1, agg1_gg, agg1_cg, W1_gg, b1_gg, eps1_gg, W1_cg, b1_cg,
                eps1_cg)

    return _head(xg2, W_out, b_out)
